# baseline (speedup 1.0000x reference)
import functools
import jax
import jax.numpy as jnp
from jax import lax
from jax.experimental import pallas as pl
from jax.experimental.pallas import tpu as pltpu
from jax.experimental.pallas import tpu_sc as plsc

N_NODES = 10000
N_EDGES = 320000
D_FEAT = 128
D_HID = 16
N_CLASSES = 40

NC, NS, LANES = 2, 16, 16
NW = NC * NS
CHUNK = 128
NCHUNK = -(-N_EDGES // (NW * CHUNK))
E_PAD = NW * NCHUNK * CHUNK
TRASH = N_NODES
ACC_ROWS = ((N_NODES + 1 + NS * CHUNK - 1) // (NS * CHUNK)) * (NS * CHUNK)

NBUF = 10
KLAG = 5
NDEG = 8

_ZROWS = ACC_ROWS // NS
_OUT_ROWS = N_NODES // NS

_mesh = plsc.VectorSubcoreMesh(
    core_axis_name="c", subcore_axis_name="s", num_cores=NC, num_subcores=NS
)


def _rsqrt16(x):
    i = plsc.bitcast(x, jnp.int32)
    i = 0x5F3759DF - lax.shift_right_logical(i, 1)
    y = plsc.bitcast(i, jnp.float32)
    for _ in range(3):
        y = y * (1.5 - 0.5 * x * y * y)
    return y


def _fill_zero_bufs(zbuf, zbuf1):
    def zrow(i, _):
        zbuf[i] = jnp.zeros((LANES,), jnp.float32)
        return 0

    lax.fori_loop(0, CHUNK, zrow, 0)
    for i in range(CHUNK // LANES):
        zbuf1[pl.ds(i * LANES, LANES)] = jnp.zeros((LANES,), jnp.float32)


def _fill_ones(obuf1):
    for i in range(CHUNK // LANES):
        obuf1[pl.ds(i * LANES, LANES)] = jnp.ones((LANES,), jnp.float32)


def _deg_phase(dacc, idx_d, idx_dm, obuf1, dsem):
    def s_start(buf, c):
        b = c % NDEG
        pltpu.async_copy(obuf1, dacc.at[buf.at[c]], dsem.at[b], add=True)

    def s_wait(buf, c):
        b = c % NDEG
        pltpu.make_async_copy(obuf1, dacc.at[buf.at[c]], dsem.at[b]).wait()

    for buf in (idx_d, idx_dm):
        for j in range(NDEG):
            s_start(buf, j)

        def step(j, _):
            s_wait(buf, j - NDEG)
            s_start(buf, j)
            return 0

        lax.fori_loop(NDEG, NCHUNK, step, 0)
        for c in range(NCHUNK - NDEG, NCHUNK):
            s_wait(buf, c)


def _edge_pipeline(table, acc, idx_s, idx_d, rbuf, gsem, ssem):
    def g_start(c):
        b = c % NBUF
        pltpu.async_copy(table.at[idx_s.at[c]], rbuf.at[b], gsem.at[b])

    def g_wait(c):
        b = c % NBUF
        pltpu.make_async_copy(table.at[idx_s.at[c]], rbuf.at[b], gsem.at[b]).wait()

    def s_start(c):
        b = c % NBUF
        pltpu.async_copy(rbuf.at[b], acc.at[idx_d.at[c]], ssem.at[b], add=True)

    def s_wait(c):
        b = c % NBUF
        pltpu.make_async_copy(rbuf.at[b], acc.at[idx_d.at[c]], ssem.at[b]).wait()

    for j in range(KLAG):
        g_start(j)
    for j in range(KLAG, NBUF):
        g_start(j)
        g_wait(j - KLAG)
        s_start(j - KLAG)

    def steady(j, _):
        s_wait(j - NBUF)
        g_start(j)
        g_wait(j - KLAG)
        s_start(j - KLAG)
        return 0

    lax.fori_loop(NBUF, NCHUNK, steady, 0)
    for c in range(NCHUNK - KLAG, NCHUNK):
        g_wait(c)
        s_start(c)
    for c in range(NCHUNK - NBUF, NCHUNK):
        s_wait(c)


def _common_prologue(src_hbm, dst_hbm, idx_s, idx_d, idx_dm,
                     zbuf, zbuf1, obuf1, dacc, acc, dsem):
    cid = lax.axis_index("c")
    sid = lax.axis_index("s")
    t = cid * NS + sid
    tm = (1 - cid) * NS + sid
    pltpu.sync_copy(src_hbm.at[t], idx_s)
    pltpu.sync_copy(dst_hbm.at[t], idx_d)
    pltpu.sync_copy(dst_hbm.at[tm], idx_dm)
    _fill_zero_bufs(zbuf, zbuf1)
    _fill_ones(obuf1)
    base = sid * _ZROWS
    for k in range(_ZROWS // CHUNK):
        pltpu.sync_copy(zbuf, acc.at[pl.ds(base + k * CHUNK, CHUNK)])
        pltpu.sync_copy(zbuf1, dacc.at[pl.ds(base + k * CHUNK, CHUNK)])
    plsc.subcore_barrier()
    _deg_phase(dacc, idx_d, idx_dm, obuf1, dsem)
    plsc.subcore_barrier()
    return cid, sid


def _load_deg_slice(dacc, dbuf, sid):
    base = sid * _OUT_ROWS
    abase = (base // 8) * 8
    pltpu.sync_copy(dacc.at[pl.ds(abase, _ZROWS)], dbuf.at[pl.ds(0, _ZROWS)])
    return base - abase


def _deg_bcast(dbuf, r):
    v = dbuf[pl.ds(r, LANES)]
    return jnp.broadcast_to(v[0] + 1.0, (LANES,))


_SC_SCRATCH = [
    pltpu.VMEM((NCHUNK, CHUNK), jnp.int32),
    pltpu.VMEM((NCHUNK, CHUNK), jnp.int32),
    pltpu.VMEM((NCHUNK, CHUNK), jnp.int32),
    pltpu.VMEM((NBUF, CHUNK, D_HID), jnp.float32),
    pltpu.VMEM((CHUNK, D_HID), jnp.float32),
    pltpu.VMEM((CHUNK,), jnp.float32),
    pltpu.VMEM((CHUNK,), jnp.float32),
    pltpu.VMEM((_ZROWS + LANES,), jnp.float32),
    pltpu.VMEM((_OUT_ROWS, D_HID), jnp.float32),
    pltpu.VMEM_SHARED((ACC_ROWS, D_HID), jnp.float32),
    pltpu.VMEM_SHARED((ACC_ROWS, D_HID), jnp.float32),
    pltpu.VMEM_SHARED((ACC_ROWS,), jnp.float32),
    pltpu.SemaphoreType.DMA((NBUF,)),
    pltpu.SemaphoreType.DMA((NBUF,)),
    pltpu.SemaphoreType.DMA((NDEG,)),
]


@functools.partial(
    pl.kernel,
    out_type=jax.ShapeDtypeStruct((NC, N_NODES, D_HID), jnp.float32),
    mesh=_mesh,
    scratch_types=_SC_SCRATCH,
    compiler_params=pltpu.CompilerParams(
        use_tc_tiling_on_sc=False, needs_layout_passes=False
    ),
)
def _sc_layer1(src_hbm, dst_hbm, h1_hbm, out_hbm,
               idx_s, idx_d, idx_dm, rbuf, zbuf, zbuf1, obuf1, dbuf, fbuf,
               table, acc, dacc, gsem, ssem, dsem):
    cid, sid = _common_prologue(src_hbm, dst_hbm, idx_s, idx_d, idx_dm,
                                zbuf, zbuf1, obuf1, dacc, acc, dsem)
    base = sid * _OUT_ROWS
    off = _load_deg_slice(dacc, dbuf, sid)
    pltpu.sync_copy(h1_hbm.at[pl.ds(base, _OUT_ROWS)], fbuf)

    def prep_row(r, _):
        dinv = _rsqrt16(_deg_bcast(dbuf, r + off))
        fbuf[r] = fbuf[r] * dinv
        return 0

    lax.fori_loop(0, _OUT_ROWS, prep_row, 0)
    pltpu.sync_copy(fbuf, table.at[pl.ds(base, _OUT_ROWS)])
    plsc.subcore_barrier()
    _edge_pipeline(table, acc, idx_s, idx_d, rbuf, gsem, ssem)
    plsc.subcore_barrier()
    pltpu.sync_copy(
        acc.at[pl.ds(base, _OUT_ROWS)],
        out_hbm.at[cid].at[pl.ds(base, _OUT_ROWS)],
    )


@functools.partial(
    pl.kernel,
    out_type=(
        jax.ShapeDtypeStruct((NC, N_NODES, D_HID), jnp.float32),
        jax.ShapeDtypeStruct((N_NODES, D_HID), jnp.float32),
        jax.ShapeDtypeStruct((N_NODES, D_HID), jnp.float32),
    ),
    mesh=_mesh,
    scratch_types=_SC_SCRATCH + [
        pltpu.VMEM((_OUT_ROWS, D_HID), jnp.float32),
        pltpu.VMEM((_OUT_ROWS, D_HID), jnp.float32),
        pltpu.VMEM((_OUT_ROWS, D_HID), jnp.float32),
        pltpu.VMEM((_OUT_ROWS, D_HID), jnp.float32),
        pltpu.VMEM((LANES,), jnp.float32),
    ],
    compiler_params=pltpu.CompilerParams(
        use_tc_tiling_on_sc=False, needs_layout_passes=False
    ),
)
def _sc_layer2(src_hbm, dst_hbm, h1_hbm, p1_hbm, b1_hbm,
               out_hbm, s2_hbm, dv_hbm,
               idx_s, idx_d, idx_dm, rbuf, zbuf, zbuf1, obuf1, dbuf, fbuf,
               table, acc, dacc, gsem, ssem, dsem,
               p0buf, p1buf, s2buf, dvbuf, b1buf):
    cid, sid = _common_prologue(src_hbm, dst_hbm, idx_s, idx_d, idx_dm,
                                zbuf, zbuf1, obuf1, dacc, acc, dsem)
    base = sid * _OUT_ROWS
    off = _load_deg_slice(dacc, dbuf, sid)
    pltpu.sync_copy(h1_hbm.at[pl.ds(base, _OUT_ROWS)], fbuf)
    pltpu.sync_copy(p1_hbm.at[0].at[pl.ds(base, _OUT_ROWS)], p0buf)
    pltpu.sync_copy(p1_hbm.at[1].at[pl.ds(base, _OUT_ROWS)], p1buf)
    pltpu.sync_copy(b1_hbm, b1buf)
    b1v = b1buf[...]

    def prep_row(r, _):
        dinv = _rsqrt16(_deg_bcast(dbuf, r + off))
        h = dinv * (p0buf[r] + p1buf[r]) + dinv * dinv * fbuf[r] + b1v
        h = jnp.maximum(h, 0.0)
        fbuf[r] = h * dinv
        s2buf[r] = h * dinv * dinv
        dvbuf[r] = dinv
        return 0

    lax.fori_loop(0, _OUT_ROWS, prep_row, 0)
    pltpu.sync_copy(fbuf, table.at[pl.ds(base, _OUT_ROWS)])
    pltpu.sync_copy(s2buf, s2_hbm.at[pl.ds(base, _OUT_ROWS)])
    pltpu.sync_copy(dvbuf, dv_hbm.at[pl.ds(base, _OUT_ROWS)])
    plsc.subcore_barrier()
    _edge_pipeline(table, acc, idx_s, idx_d, rbuf, gsem, ssem)
    plsc.subcore_barrier()
    pltpu.sync_copy(
        acc.at[pl.ds(base, _OUT_ROWS)],
        out_hbm.at[cid].at[pl.ds(base, _OUT_ROWS)],
    )


def _tc_matmul1(x_ref, w_ref, o_ref):
    o_ref[...] = jnp.dot(x_ref[...], w_ref[...], preferred_element_type=jnp.float32)


def _tc_final(aggp_ref, s2_ref, d_ref, w2_ref, b2_ref, o_ref):
    z = (aggp_ref[0] + aggp_ref[1]) * d_ref[...] + s2_ref[...]
    logits = jnp.dot(z, w2_ref[...], preferred_element_type=jnp.float32) + b2_ref[...]
    m = jnp.max(logits, axis=1, keepdims=True)
    e = jnp.exp(logits - m)
    o_ref[...] = logits - m - jnp.log(jnp.sum(e, axis=1, keepdims=True))


def _f32(shape):
    return jax.ShapeDtypeStruct(shape, jnp.float32)


@jax.jit
def kernel(x, edge_index, W1, b1, W2, b2):
    src = edge_index[0].astype(jnp.int32)
    dst = edge_index[1].astype(jnp.int32)
    pad = E_PAD - N_EDGES
    src_p = jnp.concatenate([src, jnp.zeros((pad,), jnp.int32)]).reshape(
        NW, NCHUNK, CHUNK
    )
    dst_p = jnp.concatenate([dst, jnp.full((pad,), TRASH, jnp.int32)]).reshape(
        NW, NCHUNK, CHUNK
    )

    h1 = pl.pallas_call(_tc_matmul1, out_shape=_f32((N_NODES, D_HID)))(x, W1)
    agg1 = _sc_layer1(src_p, dst_p, h1)
    agg2, s2, d = _sc_layer2(src_p, dst_p, h1, agg1, b1)
    out = pl.pallas_call(
        _tc_final,
        out_shape=_f32((N_NODES, N_CLASSES)),
    )(agg2, s2, d, W2, b2.reshape(1, N_CLASSES))
    return out


# speedup vs baseline: 1.1503x; 1.1503x over previous
import functools
import jax
import jax.numpy as jnp
from jax import lax
from jax.experimental import pallas as pl
from jax.experimental.pallas import tpu as pltpu
from jax.experimental.pallas import tpu_sc as plsc

N_NODES = 10000
N_EDGES = 320000
D_FEAT = 128
D_HID = 16
N_CLASSES = 40

NC, NS, LANES = 2, 16, 16
NW = NC * NS
CHUNK = 128
E_TILE = N_EDGES // NW
NFULL = E_TILE // CHUNK
TAIL = E_TILE - NFULL * CHUNK
ACC_ROWS = ((N_NODES + NS * CHUNK - 1) // (NS * CHUNK)) * (NS * CHUNK)

NBUF = 10
KLAG = 5
NDEG = 8

_ZROWS = ACC_ROWS // NS
_OUT_ROWS = N_NODES // NS

_mesh = plsc.VectorSubcoreMesh(
    core_axis_name="c", subcore_axis_name="s", num_cores=NC, num_subcores=NS
)


def _rsqrt16(x):
    i = plsc.bitcast(x, jnp.int32)
    i = 0x5F3759DF - lax.shift_right_logical(i, 1)
    y = plsc.bitcast(i, jnp.float32)
    for _ in range(3):
        y = y * (1.5 - 0.5 * x * y * y)
    return y


def _fill_zero_bufs(zbuf, zbuf1):
    def zrow(i, _):
        zbuf[i] = jnp.zeros((LANES,), jnp.float32)
        return 0

    lax.fori_loop(0, CHUNK, zrow, 0)
    for i in range(CHUNK // LANES):
        zbuf1[pl.ds(i * LANES, LANES)] = jnp.zeros((LANES,), jnp.float32)


def _fill_ones(obuf1):
    for i in range(CHUNK // LANES):
        obuf1[pl.ds(i * LANES, LANES)] = jnp.ones((LANES,), jnp.float32)


def _deg_phase(dacc, idx_d, idx_dm, obuf1, dsem):
    def s_start(buf, c, n=CHUNK):
        b = c % NDEG
        pltpu.async_copy(obuf1.at[pl.ds(0, n)],
                         dacc.at[buf.at[pl.ds(c * CHUNK, n)]],
                         dsem.at[b], add=True)

    def s_wait(buf, c, n=CHUNK):
        b = c % NDEG
        pltpu.make_async_copy(obuf1.at[pl.ds(0, n)],
                              dacc.at[buf.at[pl.ds(c * CHUNK, n)]],
                              dsem.at[b]).wait()

    for buf in (idx_d, idx_dm):
        for j in range(NDEG):
            s_start(buf, j)

        def step(j, _):
            s_wait(buf, j - NDEG)
            s_start(buf, j)
            return 0

        lax.fori_loop(NDEG, NFULL, step, 0)
        for c in range(NFULL - NDEG, NFULL):
            s_wait(buf, c)
        s_start(buf, NFULL, TAIL)
        s_wait(buf, NFULL, TAIL)


def _edge_pipeline(table, acc, idx_s, idx_d, rbuf, gsem, ssem):
    def g_start(c, n=CHUNK):
        b = c % NBUF
        pltpu.async_copy(table.at[idx_s.at[pl.ds(c * CHUNK, n)]],
                         rbuf.at[b].at[pl.ds(0, n)], gsem.at[b])

    def g_wait(c, n=CHUNK):
        b = c % NBUF
        pltpu.make_async_copy(table.at[idx_s.at[pl.ds(c * CHUNK, n)]],
                              rbuf.at[b].at[pl.ds(0, n)], gsem.at[b]).wait()

    def s_start(c, n=CHUNK):
        b = c % NBUF
        pltpu.async_copy(rbuf.at[b].at[pl.ds(0, n)],
                         acc.at[idx_d.at[pl.ds(c * CHUNK, n)]],
                         ssem.at[b], add=True)

    def s_wait(c, n=CHUNK):
        b = c % NBUF
        pltpu.make_async_copy(rbuf.at[b].at[pl.ds(0, n)],
                              acc.at[idx_d.at[pl.ds(c * CHUNK, n)]],
                              ssem.at[b]).wait()

    for j in range(KLAG):
        g_start(j)
    for j in range(KLAG, NBUF):
        g_start(j)
        g_wait(j - KLAG)
        s_start(j - KLAG)

    def steady(j, _):
        s_wait(j - NBUF)
        g_start(j)
        g_wait(j - KLAG)
        s_start(j - KLAG)
        return 0

    lax.fori_loop(NBUF, NFULL, steady, 0)
    for c in range(NFULL - KLAG, NFULL):
        g_wait(c)
        s_start(c)
    for c in range(NFULL - NBUF, NFULL):
        s_wait(c)
    g_start(NFULL, TAIL)
    g_wait(NFULL, TAIL)
    s_start(NFULL, TAIL)
    s_wait(NFULL, TAIL)


def _common_prologue(ei_hbm, idx_s, idx_d, idx_dm,
                     zbuf, zbuf1, obuf1, dacc, acc, dsem):
    cid = lax.axis_index("c")
    sid = lax.axis_index("s")
    t = cid * NS + sid
    tm = (1 - cid) * NS + sid
    pltpu.sync_copy(ei_hbm.at[0].at[pl.ds(t * E_TILE, E_TILE)], idx_s)
    pltpu.sync_copy(ei_hbm.at[1].at[pl.ds(t * E_TILE, E_TILE)], idx_d)
    pltpu.sync_copy(ei_hbm.at[1].at[pl.ds(tm * E_TILE, E_TILE)], idx_dm)
    _fill_zero_bufs(zbuf, zbuf1)
    _fill_ones(obuf1)
    base = sid * _ZROWS
    for k in range(_ZROWS // CHUNK):
        pltpu.sync_copy(zbuf, acc.at[pl.ds(base + k * CHUNK, CHUNK)])
        pltpu.sync_copy(zbuf1, dacc.at[pl.ds(base + k * CHUNK, CHUNK)])
    plsc.subcore_barrier()
    _deg_phase(dacc, idx_d, idx_dm, obuf1, dsem)
    plsc.subcore_barrier()
    return cid, sid


def _load_deg_slice(dacc, dbuf, sid):
    base = sid * _OUT_ROWS
    abase = (base // 8) * 8
    pltpu.sync_copy(dacc.at[pl.ds(abase, _ZROWS)], dbuf.at[pl.ds(0, _ZROWS)])
    return base - abase


def _deg_bcast(dbuf, r):
    v = dbuf[pl.ds(r, LANES)]
    return jnp.broadcast_to(v[0] + 1.0, (LANES,))


_SC_SCRATCH = [
    pltpu.VMEM((E_TILE,), jnp.int32),
    pltpu.VMEM((E_TILE,), jnp.int32),
    pltpu.VMEM((E_TILE,), jnp.int32),
    pltpu.VMEM((NBUF, CHUNK, D_HID), jnp.float32),
    pltpu.VMEM((CHUNK, D_HID), jnp.float32),
    pltpu.VMEM((CHUNK,), jnp.float32),
    pltpu.VMEM((CHUNK,), jnp.float32),
    pltpu.VMEM((_ZROWS + LANES,), jnp.float32),
    pltpu.VMEM((_OUT_ROWS, D_HID), jnp.float32),
    pltpu.VMEM_SHARED((ACC_ROWS, D_HID), jnp.float32),
    pltpu.VMEM_SHARED((ACC_ROWS, D_HID), jnp.float32),
    pltpu.VMEM_SHARED((ACC_ROWS,), jnp.float32),
    pltpu.SemaphoreType.DMA((NBUF,)),
    pltpu.SemaphoreType.DMA((NBUF,)),
    pltpu.SemaphoreType.DMA((NDEG,)),
]


@functools.partial(
    pl.kernel,
    out_type=jax.ShapeDtypeStruct((NC, N_NODES, D_HID), jnp.float32),
    mesh=_mesh,
    scratch_types=_SC_SCRATCH,
    compiler_params=pltpu.CompilerParams(
        use_tc_tiling_on_sc=False, needs_layout_passes=False
    ),
)
def _sc_layer1(ei_hbm, h1_hbm, out_hbm,
               idx_s, idx_d, idx_dm, rbuf, zbuf, zbuf1, obuf1, dbuf, fbuf,
               table, acc, dacc, gsem, ssem, dsem):
    cid, sid = _common_prologue(ei_hbm, idx_s, idx_d, idx_dm,
                                zbuf, zbuf1, obuf1, dacc, acc, dsem)
    base = sid * _OUT_ROWS
    off = _load_deg_slice(dacc, dbuf, sid)
    pltpu.sync_copy(h1_hbm.at[pl.ds(base, _OUT_ROWS)], fbuf)

    def prep_row(r, _):
        dinv = _rsqrt16(_deg_bcast(dbuf, r + off))
        fbuf[r] = fbuf[r] * dinv
        return 0

    lax.fori_loop(0, _OUT_ROWS, prep_row, 0)
    pltpu.sync_copy(fbuf, table.at[pl.ds(base, _OUT_ROWS)])
    plsc.subcore_barrier()
    _edge_pipeline(table, acc, idx_s, idx_d, rbuf, gsem, ssem)
    plsc.subcore_barrier()
    pltpu.sync_copy(
        acc.at[pl.ds(base, _OUT_ROWS)],
        out_hbm.at[cid].at[pl.ds(base, _OUT_ROWS)],
    )


@functools.partial(
    pl.kernel,
    out_type=(
        jax.ShapeDtypeStruct((NC, N_NODES, D_HID), jnp.float32),
        jax.ShapeDtypeStruct((N_NODES, D_HID), jnp.float32),
        jax.ShapeDtypeStruct((N_NODES, D_HID), jnp.float32),
    ),
    mesh=_mesh,
    scratch_types=_SC_SCRATCH + [
        pltpu.VMEM((_OUT_ROWS, D_HID), jnp.float32),
        pltpu.VMEM((_OUT_ROWS, D_HID), jnp.float32),
        pltpu.VMEM((_OUT_ROWS, D_HID), jnp.float32),
        pltpu.VMEM((_OUT_ROWS, D_HID), jnp.float32),
        pltpu.VMEM((LANES,), jnp.float32),
    ],
    compiler_params=pltpu.CompilerParams(
        use_tc_tiling_on_sc=False, needs_layout_passes=False
    ),
)
def _sc_layer2(ei_hbm, h1_hbm, p1_hbm, b1_hbm,
               out_hbm, s2_hbm, dv_hbm,
               idx_s, idx_d, idx_dm, rbuf, zbuf, zbuf1, obuf1, dbuf, fbuf,
               table, acc, dacc, gsem, ssem, dsem,
               p0buf, p1buf, s2buf, dvbuf, b1buf):
    cid, sid = _common_prologue(ei_hbm, idx_s, idx_d, idx_dm,
                                zbuf, zbuf1, obuf1, dacc, acc, dsem)
    base = sid * _OUT_ROWS
    off = _load_deg_slice(dacc, dbuf, sid)
    pltpu.sync_copy(h1_hbm.at[pl.ds(base, _OUT_ROWS)], fbuf)
    pltpu.sync_copy(p1_hbm.at[0].at[pl.ds(base, _OUT_ROWS)], p0buf)
    pltpu.sync_copy(p1_hbm.at[1].at[pl.ds(base, _OUT_ROWS)], p1buf)
    pltpu.sync_copy(b1_hbm, b1buf)
    b1v = b1buf[...]

    def prep_row(r, _):
        dinv = _rsqrt16(_deg_bcast(dbuf, r + off))
        h = dinv * (p0buf[r] + p1buf[r]) + dinv * dinv * fbuf[r] + b1v
        h = jnp.maximum(h, 0.0)
        fbuf[r] = h * dinv
        s2buf[r] = h * dinv * dinv
        dvbuf[r] = dinv
        return 0

    lax.fori_loop(0, _OUT_ROWS, prep_row, 0)
    pltpu.sync_copy(fbuf, table.at[pl.ds(base, _OUT_ROWS)])
    pltpu.sync_copy(s2buf, s2_hbm.at[pl.ds(base, _OUT_ROWS)])
    pltpu.sync_copy(dvbuf, dv_hbm.at[pl.ds(base, _OUT_ROWS)])
    plsc.subcore_barrier()
    _edge_pipeline(table, acc, idx_s, idx_d, rbuf, gsem, ssem)
    plsc.subcore_barrier()
    pltpu.sync_copy(
        acc.at[pl.ds(base, _OUT_ROWS)],
        out_hbm.at[cid].at[pl.ds(base, _OUT_ROWS)],
    )


def _tc_matmul1(x_ref, w_ref, o_ref):
    o_ref[...] = jnp.dot(x_ref[...], w_ref[...], preferred_element_type=jnp.float32)


def _tc_final(aggp_ref, s2_ref, d_ref, w2_ref, b2_ref, o_ref):
    z = (aggp_ref[0] + aggp_ref[1]) * d_ref[...] + s2_ref[...]
    logits = jnp.dot(z, w2_ref[...], preferred_element_type=jnp.float32) + b2_ref[...]
    m = jnp.max(logits, axis=1, keepdims=True)
    e = jnp.exp(logits - m)
    o_ref[...] = logits - m - jnp.log(jnp.sum(e, axis=1, keepdims=True))


def _f32(shape):
    return jax.ShapeDtypeStruct(shape, jnp.float32)


@jax.jit
def kernel(x, edge_index, W1, b1, W2, b2):
    ei = edge_index.astype(jnp.int32)
    h1 = pl.pallas_call(_tc_matmul1, out_shape=_f32((N_NODES, D_HID)))(x, W1)
    agg1 = _sc_layer1(ei, h1)
    agg2, s2, d = _sc_layer2(ei, h1, agg1, b1)
    out = pl.pallas_call(
        _tc_final,
        out_shape=_f32((N_NODES, N_CLASSES)),
    )(agg2, s2, d, W2, b2.reshape(1, N_CLASSES))
    return out


# speedup vs baseline: 1.2070x; 1.0493x over previous
import functools
import jax
import jax.numpy as jnp
from jax import lax
from jax.experimental import pallas as pl
from jax.experimental.pallas import tpu as pltpu
from jax.experimental.pallas import tpu_sc as plsc

N_NODES = 10000
N_EDGES = 320000
D_FEAT = 128
D_HID = 16
N_CLASSES = 40

NC, NS, LANES = 2, 16, 16
NW = NC * NS
CHUNK = 128
E_TILE = N_EDGES // NW
NFULL = E_TILE // CHUNK
TAIL = E_TILE - NFULL * CHUNK
ACC_ROWS = ((N_NODES + NS * CHUNK - 1) // (NS * CHUNK)) * (NS * CHUNK)

NBUF = 10
KLAG = 5
NDEG = 8

_ZROWS = ACC_ROWS // NS
_OUT_ROWS = N_NODES // NS

_mesh = plsc.VectorSubcoreMesh(
    core_axis_name="c", subcore_axis_name="s", num_cores=NC, num_subcores=NS
)


def _rsqrt16(x):
    i = plsc.bitcast(x, jnp.int32)
    i = 0x5F3759DF - lax.shift_right_logical(i, 1)
    y = plsc.bitcast(i, jnp.float32)
    for _ in range(3):
        y = y * (1.5 - 0.5 * x * y * y)
    return y


def _fill_zero_bufs(zbuf, zbuf1):
    def zrow(i, _):
        zbuf[i] = jnp.zeros((LANES,), jnp.float32)
        return 0

    lax.fori_loop(0, CHUNK, zrow, 0)
    for i in range(CHUNK // LANES):
        zbuf1[pl.ds(i * LANES, LANES)] = jnp.zeros((LANES,), jnp.float32)


def _fill_ones(obuf1):
    for i in range(CHUNK // LANES):
        obuf1[pl.ds(i * LANES, LANES)] = jnp.ones((LANES,), jnp.float32)


def _deg_phase(dacc, idx_d, idx_dm, obuf1, dsem):
    def s_start(buf, c, n=CHUNK):
        b = c % NDEG
        pltpu.async_copy(obuf1.at[pl.ds(0, n)],
                         dacc.at[buf.at[pl.ds(c * CHUNK, n)]],
                         dsem.at[b], add=True)

    def s_wait(buf, c, n=CHUNK):
        b = c % NDEG
        pltpu.make_async_copy(obuf1.at[pl.ds(0, n)],
                              dacc.at[buf.at[pl.ds(c * CHUNK, n)]],
                              dsem.at[b]).wait()

    for buf in (idx_d, idx_dm):
        for j in range(NDEG):
            s_start(buf, j)

        def step(j, _):
            s_wait(buf, j - NDEG)
            s_start(buf, j)
            return 0

        lax.fori_loop(NDEG, NFULL, step, 0)
        for c in range(NFULL - NDEG, NFULL):
            s_wait(buf, c)
        s_start(buf, NFULL, TAIL)
        s_wait(buf, NFULL, TAIL)


def _edge_pipeline(table, acc, idx_s, idx_d, rbuf, gsem, ssem):
    def g_start(c, n=CHUNK):
        b = c % NBUF
        pltpu.async_copy(table.at[idx_s.at[pl.ds(c * CHUNK, n)]],
                         rbuf.at[b].at[pl.ds(0, n)], gsem.at[b])

    def g_wait(c, n=CHUNK):
        b = c % NBUF
        pltpu.make_async_copy(table.at[idx_s.at[pl.ds(c * CHUNK, n)]],
                              rbuf.at[b].at[pl.ds(0, n)], gsem.at[b]).wait()

    def s_start(c, n=CHUNK):
        b = c % NBUF
        pltpu.async_copy(rbuf.at[b].at[pl.ds(0, n)],
                         acc.at[idx_d.at[pl.ds(c * CHUNK, n)]],
                         ssem.at[b], add=True)

    def s_wait(c, n=CHUNK):
        b = c % NBUF
        pltpu.make_async_copy(rbuf.at[b].at[pl.ds(0, n)],
                              acc.at[idx_d.at[pl.ds(c * CHUNK, n)]],
                              ssem.at[b]).wait()

    for j in range(KLAG):
        g_start(j)
    for j in range(KLAG, NBUF):
        g_start(j)
        g_wait(j - KLAG)
        s_start(j - KLAG)

    def steady(j, _):
        s_wait(j - NBUF)
        g_start(j)
        g_wait(j - KLAG)
        s_start(j - KLAG)
        return 0

    lax.fori_loop(NBUF, NFULL, steady, 0)
    for c in range(NFULL - KLAG, NFULL):
        g_wait(c)
        s_start(c)
    for c in range(NFULL - NBUF, NFULL):
        s_wait(c)
    g_start(NFULL, TAIL)
    g_wait(NFULL, TAIL)
    s_start(NFULL, TAIL)
    s_wait(NFULL, TAIL)


def _common_prologue(ei_hbm, idx_s, idx_d, idx_dm,
                     zbuf, zbuf1, obuf1, dacc, acc, dsem):
    cid = lax.axis_index("c")
    sid = lax.axis_index("s")
    t = cid * NS + sid
    tm = (1 - cid) * NS + sid
    pltpu.sync_copy(ei_hbm.at[0].at[pl.ds(t * E_TILE, E_TILE)], idx_s)
    pltpu.sync_copy(ei_hbm.at[1].at[pl.ds(t * E_TILE, E_TILE)], idx_d)
    pltpu.sync_copy(ei_hbm.at[1].at[pl.ds(tm * E_TILE, E_TILE)], idx_dm)
    _fill_zero_bufs(zbuf, zbuf1)
    _fill_ones(obuf1)
    base = sid * _ZROWS
    for k in range(_ZROWS // CHUNK):
        pltpu.sync_copy(zbuf, acc.at[pl.ds(base + k * CHUNK, CHUNK)])
        pltpu.sync_copy(zbuf1, dacc.at[pl.ds(base + k * CHUNK, CHUNK)])
    plsc.subcore_barrier()
    _deg_phase(dacc, idx_d, idx_dm, obuf1, dsem)
    plsc.subcore_barrier()
    return cid, sid


def _load_deg_slice(dacc, dbuf, sid):
    base = sid * _OUT_ROWS
    abase = (base // 8) * 8
    pltpu.sync_copy(dacc.at[pl.ds(abase, _ZROWS)], dbuf.at[pl.ds(0, _ZROWS)])
    return base - abase


def _deg_bcast(dbuf, r):
    v = dbuf[pl.ds(r, LANES)]
    return jnp.broadcast_to(v[0] + 1.0, (LANES,))


_SC_SCRATCH = [
    pltpu.VMEM((E_TILE,), jnp.int32),
    pltpu.VMEM((E_TILE,), jnp.int32),
    pltpu.VMEM((E_TILE,), jnp.int32),
    pltpu.VMEM((NBUF, CHUNK, D_HID), jnp.float32),
    pltpu.VMEM((CHUNK, D_HID), jnp.float32),
    pltpu.VMEM((CHUNK,), jnp.float32),
    pltpu.VMEM((CHUNK,), jnp.float32),
    pltpu.VMEM((_ZROWS + LANES,), jnp.float32),
    pltpu.VMEM((_OUT_ROWS, D_HID), jnp.float32),
    pltpu.VMEM_SHARED((ACC_ROWS, D_HID), jnp.float32),
    pltpu.VMEM_SHARED((ACC_ROWS, D_HID), jnp.float32),
    pltpu.VMEM_SHARED((ACC_ROWS,), jnp.float32),
    pltpu.SemaphoreType.DMA((NBUF,)),
    pltpu.SemaphoreType.DMA((NBUF,)),
    pltpu.SemaphoreType.DMA((NDEG,)),
]


@functools.partial(
    pl.kernel,
    out_type=jax.ShapeDtypeStruct((NC, N_NODES, D_HID), jnp.float32),
    mesh=_mesh,
    scratch_types=_SC_SCRATCH,
    compiler_params=pltpu.CompilerParams(
        use_tc_tiling_on_sc=False, needs_layout_passes=False
    ),
)
def _sc_layer1(ei_hbm, h1_hbm, out_hbm,
               idx_s, idx_d, idx_dm, rbuf, zbuf, zbuf1, obuf1, dbuf, fbuf,
               table, acc, dacc, gsem, ssem, dsem):
    cid, sid = _common_prologue(ei_hbm, idx_s, idx_d, idx_dm,
                                zbuf, zbuf1, obuf1, dacc, acc, dsem)
    base = sid * _OUT_ROWS
    off = _load_deg_slice(dacc, dbuf, sid)
    pltpu.sync_copy(h1_hbm.at[pl.ds(base, _OUT_ROWS)], fbuf)

    def prep_row(r, _):
        dinv = _rsqrt16(_deg_bcast(dbuf, r + off))
        fbuf[r] = fbuf[r] * dinv
        return 0

    lax.fori_loop(0, _OUT_ROWS, prep_row, 0)
    pltpu.sync_copy(fbuf, table.at[pl.ds(base, _OUT_ROWS)])
    plsc.subcore_barrier()
    _edge_pipeline(table, acc, idx_s, idx_d, rbuf, gsem, ssem)
    plsc.subcore_barrier()
    pltpu.sync_copy(
        acc.at[pl.ds(base, _OUT_ROWS)],
        out_hbm.at[cid].at[pl.ds(base, _OUT_ROWS)],
    )


@functools.partial(
    pl.kernel,
    out_type=jax.ShapeDtypeStruct((NC, N_NODES, D_HID), jnp.float32),
    mesh=_mesh,
    scratch_types=_SC_SCRATCH + [
        pltpu.VMEM((_OUT_ROWS, D_HID), jnp.float32),
        pltpu.VMEM((_OUT_ROWS, D_HID), jnp.float32),
        pltpu.VMEM((_OUT_ROWS, D_HID), jnp.float32),
        pltpu.VMEM((_OUT_ROWS, D_HID), jnp.float32),
        pltpu.VMEM((LANES,), jnp.float32),
    ],
    compiler_params=pltpu.CompilerParams(
        use_tc_tiling_on_sc=False, needs_layout_passes=False
    ),
)
def _sc_layer2(ei_hbm, h1_hbm, p1_hbm, b1_hbm, out_hbm,
               idx_s, idx_d, idx_dm, rbuf, zbuf, zbuf1, obuf1, dbuf, fbuf,
               table, acc, dacc, gsem, ssem, dsem,
               p0buf, p1buf, s2buf, dvbuf, b1buf):
    cid, sid = _common_prologue(ei_hbm, idx_s, idx_d, idx_dm,
                                zbuf, zbuf1, obuf1, dacc, acc, dsem)
    base = sid * _OUT_ROWS
    off = _load_deg_slice(dacc, dbuf, sid)
    pltpu.sync_copy(h1_hbm.at[pl.ds(base, _OUT_ROWS)], fbuf)
    pltpu.sync_copy(p1_hbm.at[0].at[pl.ds(base, _OUT_ROWS)], p0buf)
    pltpu.sync_copy(p1_hbm.at[1].at[pl.ds(base, _OUT_ROWS)], p1buf)
    pltpu.sync_copy(b1_hbm, b1buf)
    b1v = b1buf[...]

    def prep_row(r, _):
        dinv = _rsqrt16(_deg_bcast(dbuf, r + off))
        h = dinv * (p0buf[r] + p1buf[r]) + dinv * dinv * fbuf[r] + b1v
        h = jnp.maximum(h, 0.0)
        fbuf[r] = h * dinv
        s2buf[r] = h * dinv * dinv
        dvbuf[r] = dinv
        return 0

    lax.fori_loop(0, _OUT_ROWS, prep_row, 0)
    pltpu.sync_copy(fbuf, table.at[pl.ds(base, _OUT_ROWS)])
    plsc.subcore_barrier()
    _edge_pipeline(table, acc, idx_s, idx_d, rbuf, gsem, ssem)
    plsc.subcore_barrier()
    pltpu.sync_copy(acc.at[pl.ds(base, _OUT_ROWS)], p0buf)
    f = jnp.broadcast_to(
        jnp.where(cid == 0, 1.0, 0.0).astype(jnp.float32), (LANES,)
    )

    def z_row(r, _):
        p0buf[r] = dvbuf[r] * p0buf[r] + f * s2buf[r]
        return 0

    lax.fori_loop(0, _OUT_ROWS, z_row, 0)
    pltpu.sync_copy(p0buf, out_hbm.at[cid].at[pl.ds(base, _OUT_ROWS)])


def _tc_matmul1(x_ref, w_ref, o_ref):
    o_ref[...] = jnp.dot(x_ref[...], w_ref[...], preferred_element_type=jnp.float32)


def _tc_final(zp_ref, w2_ref, b2_ref, o_ref):
    z = zp_ref[0] + zp_ref[1]
    logits = jnp.dot(z, w2_ref[...], preferred_element_type=jnp.float32) + b2_ref[...]
    m = jnp.max(logits, axis=1, keepdims=True)
    e = jnp.exp(logits - m)
    o_ref[...] = logits - m - jnp.log(jnp.sum(e, axis=1, keepdims=True))


def _f32(shape):
    return jax.ShapeDtypeStruct(shape, jnp.float32)


@jax.jit
def kernel(x, edge_index, W1, b1, W2, b2):
    ei = edge_index.astype(jnp.int32)
    h1 = pl.pallas_call(_tc_matmul1, out_shape=_f32((N_NODES, D_HID)))(x, W1)
    agg1 = _sc_layer1(ei, h1)
    zp = _sc_layer2(ei, h1, agg1, b1)
    out = pl.pallas_call(
        _tc_final,
        out_shape=_f32((N_NODES, N_CLASSES)),
    )(zp, W2, b2.reshape(1, N_CLASSES))
    return out


# speedup vs baseline: 1.2537x; 1.0387x over previous
import functools
import jax
import jax.numpy as jnp
from jax import lax
from jax.experimental import pallas as pl
from jax.experimental.pallas import tpu as pltpu
from jax.experimental.pallas import tpu_sc as plsc

N_NODES = 10000
N_EDGES = 320000
D_FEAT = 128
D_HID = 16
N_CLASSES = 40

NC, NS, LANES = 2, 16, 16
NW = NC * NS
CHUNK = 128
E_TILE = N_EDGES // NW
NFULL = E_TILE // CHUNK
TAIL = E_TILE - NFULL * CHUNK
ACC_ROWS = ((N_NODES + NS * CHUNK - 1) // (NS * CHUNK)) * (NS * CHUNK)

NBUF = 6
KLAG = 3
NDEG = 8

_ZROWS = ACC_ROWS // NS
_OUT_ROWS = N_NODES // NS

_mesh = plsc.VectorSubcoreMesh(
    core_axis_name="c", subcore_axis_name="s", num_cores=NC, num_subcores=NS
)


def _rsqrt16(x):
    i = plsc.bitcast(x, jnp.int32)
    i = 0x5F3759DF - lax.shift_right_logical(i, 1)
    y = plsc.bitcast(i, jnp.float32)
    for _ in range(2):
        y = y * (1.5 - 0.5 * x * y * y)
    return y


def _fill_zero_bufs(zbuf, zbuf1):
    def zrow(i, _):
        zbuf[i] = jnp.zeros((LANES,), jnp.float32)
        return 0

    lax.fori_loop(0, CHUNK, zrow, 0)
    for i in range(CHUNK // LANES):
        zbuf1[pl.ds(i * LANES, LANES)] = jnp.zeros((LANES,), jnp.float32)


def _fill_ones(obuf1):
    for i in range(CHUNK // LANES):
        obuf1[pl.ds(i * LANES, LANES)] = jnp.ones((LANES,), jnp.float32)


def _deg_phase(dacc, idx_d, idx_dm, obuf1, dsem):
    def s_start(buf, c, n=CHUNK):
        b = c % NDEG
        pltpu.async_copy(obuf1.at[pl.ds(0, n)],
                         dacc.at[buf.at[pl.ds(c * CHUNK, n)]],
                         dsem.at[b], add=True)

    def s_wait(buf, c, n=CHUNK):
        b = c % NDEG
        pltpu.make_async_copy(obuf1.at[pl.ds(0, n)],
                              dacc.at[buf.at[pl.ds(c * CHUNK, n)]],
                              dsem.at[b]).wait()

    for buf in (idx_d, idx_dm):
        for j in range(NDEG):
            s_start(buf, j)

        def step(j, _):
            s_wait(buf, j - NDEG)
            s_start(buf, j)
            return 0

        lax.fori_loop(NDEG, NFULL, step, 0)
        for c in range(NFULL - NDEG, NFULL):
            s_wait(buf, c)
        s_start(buf, NFULL, TAIL)
        s_wait(buf, NFULL, TAIL)


def _edge_pipeline(table, acc, idx_s, idx_d, rbuf, gsem, ssem):
    def g_start(c, n=CHUNK):
        b = c % NBUF
        pltpu.async_copy(table.at[idx_s.at[pl.ds(c * CHUNK, n)]],
                         rbuf.at[b].at[pl.ds(0, n)], gsem.at[b])

    def g_wait(c, n=CHUNK):
        b = c % NBUF
        pltpu.make_async_copy(table.at[idx_s.at[pl.ds(c * CHUNK, n)]],
                              rbuf.at[b].at[pl.ds(0, n)], gsem.at[b]).wait()

    def s_start(c, n=CHUNK):
        b = c % NBUF
        pltpu.async_copy(rbuf.at[b].at[pl.ds(0, n)],
                         acc.at[idx_d.at[pl.ds(c * CHUNK, n)]],
                         ssem.at[b], add=True)

    def s_wait(c, n=CHUNK):
        b = c % NBUF
        pltpu.make_async_copy(rbuf.at[b].at[pl.ds(0, n)],
                              acc.at[idx_d.at[pl.ds(c * CHUNK, n)]],
                              ssem.at[b]).wait()

    for j in range(KLAG):
        g_start(j)
    for j in range(KLAG, NBUF):
        g_start(j)
        g_wait(j - KLAG)
        s_start(j - KLAG)

    def steady(j, _):
        s_wait(j - NBUF)
        g_start(j)
        g_wait(j - KLAG)
        s_start(j - KLAG)
        return 0

    lax.fori_loop(NBUF, NFULL, steady, 0)
    for c in range(NFULL - KLAG, NFULL):
        g_wait(c)
        s_start(c)
    for c in range(NFULL - NBUF, NFULL):
        s_wait(c)
    g_start(NFULL, TAIL)
    g_wait(NFULL, TAIL)
    s_start(NFULL, TAIL)
    s_wait(NFULL, TAIL)


def _common_prologue(ei_hbm, idx_s, idx_d, idx_dm,
                     zbuf, zbuf1, obuf1, dacc, acc, dsem):
    cid = lax.axis_index("c")
    sid = lax.axis_index("s")
    t = cid * NS + sid
    tm = (1 - cid) * NS + sid
    pltpu.sync_copy(ei_hbm.at[0].at[pl.ds(t * E_TILE, E_TILE)], idx_s)
    pltpu.sync_copy(ei_hbm.at[1].at[pl.ds(t * E_TILE, E_TILE)], idx_d)
    pltpu.sync_copy(ei_hbm.at[1].at[pl.ds(tm * E_TILE, E_TILE)], idx_dm)
    _fill_zero_bufs(zbuf, zbuf1)
    _fill_ones(obuf1)
    base = sid * _ZROWS
    for k in range(_ZROWS // CHUNK):
        pltpu.sync_copy(zbuf, acc.at[pl.ds(base + k * CHUNK, CHUNK)])
        pltpu.sync_copy(zbuf1, dacc.at[pl.ds(base + k * CHUNK, CHUNK)])
    plsc.subcore_barrier()
    _deg_phase(dacc, idx_d, idx_dm, obuf1, dsem)
    plsc.subcore_barrier()
    return cid, sid


def _load_deg_slice(dacc, dbuf, sid):
    base = sid * _OUT_ROWS
    abase = (base // 8) * 8
    pltpu.sync_copy(dacc.at[pl.ds(abase, _ZROWS)], dbuf.at[pl.ds(0, _ZROWS)])
    return base - abase


def _deg_bcast(dbuf, r):
    v = dbuf[pl.ds(r, LANES)]
    return jnp.broadcast_to(v[0] + 1.0, (LANES,))


_SC_SCRATCH = [
    pltpu.VMEM((E_TILE,), jnp.int32),
    pltpu.VMEM((E_TILE,), jnp.int32),
    pltpu.VMEM((E_TILE,), jnp.int32),
    pltpu.VMEM((NBUF, CHUNK, D_HID), jnp.float32),
    pltpu.VMEM((CHUNK, D_HID), jnp.float32),
    pltpu.VMEM((CHUNK,), jnp.float32),
    pltpu.VMEM((CHUNK,), jnp.float32),
    pltpu.VMEM((_ZROWS + LANES,), jnp.float32),
    pltpu.VMEM((_OUT_ROWS, D_HID), jnp.float32),
    pltpu.VMEM_SHARED((ACC_ROWS, D_HID), jnp.float32),
    pltpu.VMEM_SHARED((ACC_ROWS, D_HID), jnp.float32),
    pltpu.VMEM_SHARED((ACC_ROWS,), jnp.float32),
    pltpu.SemaphoreType.DMA((NBUF,)),
    pltpu.SemaphoreType.DMA((NBUF,)),
    pltpu.SemaphoreType.DMA((NDEG,)),
]


@functools.partial(
    pl.kernel,
    out_type=jax.ShapeDtypeStruct((NC, N_NODES, D_HID), jnp.float32),
    mesh=_mesh,
    scratch_types=_SC_SCRATCH,
    compiler_params=pltpu.CompilerParams(
        use_tc_tiling_on_sc=False, needs_layout_passes=False
    ),
)
def _sc_layer1(ei_hbm, h1_hbm, out_hbm,
               idx_s, idx_d, idx_dm, rbuf, zbuf, zbuf1, obuf1, dbuf, fbuf,
               table, acc, dacc, gsem, ssem, dsem):
    cid, sid = _common_prologue(ei_hbm, idx_s, idx_d, idx_dm,
                                zbuf, zbuf1, obuf1, dacc, acc, dsem)
    base = sid * _OUT_ROWS
    off = _load_deg_slice(dacc, dbuf, sid)
    pltpu.sync_copy(h1_hbm.at[pl.ds(base, _OUT_ROWS)], fbuf)

    def prep_row(r, _):
        dinv = _rsqrt16(_deg_bcast(dbuf, r + off))
        fbuf[r] = fbuf[r] * dinv
        return 0

    lax.fori_loop(0, _OUT_ROWS, prep_row, 0)
    pltpu.sync_copy(fbuf, table.at[pl.ds(base, _OUT_ROWS)])
    plsc.subcore_barrier()
    _edge_pipeline(table, acc, idx_s, idx_d, rbuf, gsem, ssem)
    plsc.subcore_barrier()
    pltpu.sync_copy(
        acc.at[pl.ds(base, _OUT_ROWS)],
        out_hbm.at[cid].at[pl.ds(base, _OUT_ROWS)],
    )


@functools.partial(
    pl.kernel,
    out_type=jax.ShapeDtypeStruct((NC, N_NODES, D_HID), jnp.float32),
    mesh=_mesh,
    scratch_types=_SC_SCRATCH + [
        pltpu.VMEM((_OUT_ROWS, D_HID), jnp.float32),
        pltpu.VMEM((_OUT_ROWS, D_HID), jnp.float32),
        pltpu.VMEM((_OUT_ROWS, D_HID), jnp.float32),
        pltpu.VMEM((_OUT_ROWS, D_HID), jnp.float32),
        pltpu.VMEM((LANES,), jnp.float32),
    ],
    compiler_params=pltpu.CompilerParams(
        use_tc_tiling_on_sc=False, needs_layout_passes=False
    ),
)
def _sc_layer2(ei_hbm, h1_hbm, p1_hbm, b1_hbm, out_hbm,
               idx_s, idx_d, idx_dm, rbuf, zbuf, zbuf1, obuf1, dbuf, fbuf,
               table, acc, dacc, gsem, ssem, dsem,
               p0buf, p1buf, s2buf, dvbuf, b1buf):
    cid, sid = _common_prologue(ei_hbm, idx_s, idx_d, idx_dm,
                                zbuf, zbuf1, obuf1, dacc, acc, dsem)
    base = sid * _OUT_ROWS
    off = _load_deg_slice(dacc, dbuf, sid)
    pltpu.sync_copy(h1_hbm.at[pl.ds(base, _OUT_ROWS)], fbuf)
    pltpu.sync_copy(p1_hbm.at[0].at[pl.ds(base, _OUT_ROWS)], p0buf)
    pltpu.sync_copy(p1_hbm.at[1].at[pl.ds(base, _OUT_ROWS)], p1buf)
    pltpu.sync_copy(b1_hbm, b1buf)
    b1v = b1buf[...]

    def prep_row(r, _):
        dinv = _rsqrt16(_deg_bcast(dbuf, r + off))
        h = dinv * (p0buf[r] + p1buf[r]) + dinv * dinv * fbuf[r] + b1v
        h = jnp.maximum(h, 0.0)
        fbuf[r] = h * dinv
        s2buf[r] = h * dinv * dinv
        dvbuf[r] = dinv
        return 0

    lax.fori_loop(0, _OUT_ROWS, prep_row, 0)
    pltpu.sync_copy(fbuf, table.at[pl.ds(base, _OUT_ROWS)])
    plsc.subcore_barrier()
    _edge_pipeline(table, acc, idx_s, idx_d, rbuf, gsem, ssem)
    plsc.subcore_barrier()
    pltpu.sync_copy(acc.at[pl.ds(base, _OUT_ROWS)], p0buf)
    f = jnp.broadcast_to(
        jnp.where(cid == 0, 1.0, 0.0).astype(jnp.float32), (LANES,)
    )

    def z_row(r, _):
        p0buf[r] = dvbuf[r] * p0buf[r] + f * s2buf[r]
        return 0

    lax.fori_loop(0, _OUT_ROWS, z_row, 0)
    pltpu.sync_copy(p0buf, out_hbm.at[cid].at[pl.ds(base, _OUT_ROWS)])


def _tc_matmul1(x_ref, w_ref, o_ref):
    o_ref[...] = jnp.dot(x_ref[...], w_ref[...], preferred_element_type=jnp.float32)


def _tc_final(zp_ref, w2_ref, b2_ref, o_ref):
    z = zp_ref[0] + zp_ref[1]
    logits = jnp.dot(z, w2_ref[...], preferred_element_type=jnp.float32) + b2_ref[...]
    m = jnp.max(logits, axis=1, keepdims=True)
    e = jnp.exp(logits - m)
    o_ref[...] = logits - m - jnp.log(jnp.sum(e, axis=1, keepdims=True))


def _f32(shape):
    return jax.ShapeDtypeStruct(shape, jnp.float32)


@jax.jit
def kernel(x, edge_index, W1, b1, W2, b2):
    ei = edge_index.astype(jnp.int32)
    h1 = pl.pallas_call(_tc_matmul1, out_shape=_f32((N_NODES, D_HID)))(x, W1)
    agg1 = _sc_layer1(ei, h1)
    zp = _sc_layer2(ei, h1, agg1, b1)
    out = pl.pallas_call(
        _tc_final,
        out_shape=_f32((N_NODES, N_CLASSES)),
    )(zp, W2, b2.reshape(1, N_CLASSES))
    return out
